# baseline (speedup 1.0000x reference)
import functools

import jax
import jax.numpy as jnp
from jax import lax
from jax.experimental import pallas as pl
from jax.experimental.pallas import tpu as pltpu
from jax.experimental.pallas import tpu_sc as plsc

C, D = 64, 768
CB = 8
C_SPLIT = 16
LANES = 16
NW = 16
SC_ROWS = C - C_SPLIT
ROWS_PER_W = 4
ACTIVE_W = SC_ROWS // ROWS_PER_W


def _bias_body(cht, rgt, bc, wm, bm, wcnt, bcnt, out, rows_v, small_v, sem):
    wid = lax.axis_index("s")

    @pl.when(wid < ACTIVE_W)
    def _():
        _bias_worker(cht, rgt, bc, wm, bm, wcnt, bcnt, out, rows_v, small_v,
                     sem, wid)


def _bias_worker(cht, rgt, bc, wm, bm, wcnt, bcnt, out, rows_v, small_v, sem,
                 wid):
    base = C_SPLIT + wid * ROWS_PER_W
    copies = [
        pltpu.async_copy(cht.at[pl.ds(base, ROWS_PER_W)], rows_v, sem),
        pltpu.async_copy(rgt.at[0], small_v.at[pl.ds(0 * D, D)], sem),
        pltpu.async_copy(bc, small_v.at[pl.ds(1 * D, D)], sem),
        pltpu.async_copy(wm.at[0], small_v.at[pl.ds(2 * D, D)], sem),
        pltpu.async_copy(bm, small_v.at[pl.ds(3 * D, D)], sem),
        pltpu.async_copy(wcnt.at[0], small_v.at[pl.ds(4 * D, D)], sem),
        pltpu.async_copy(bcnt, small_v.at[pl.ds(5 * D, D)], sem),
    ]
    for cp in copies:
        cp.wait()

    def _chunk(j, carry):
        o = j * LANES
        const = (small_v[pl.ds(0 * D + o, LANES)]
                 + small_v[pl.ds(1 * D + o, LANES)]
                 + small_v[pl.ds(2 * D + o, LANES)]
                 + small_v[pl.ds(3 * D + o, LANES)]
                 + small_v[pl.ds(4 * D + o, LANES)]
                 + small_v[pl.ds(5 * D + o, LANES)])
        for r in range(ROWS_PER_W):
            rows_v[r, pl.ds(o, LANES)] = rows_v[r, pl.ds(o, LANES)] + const
        return carry

    lax.fori_loop(0, D // LANES, _chunk, 0)
    pltpu.sync_copy(rows_v, out.at[pl.ds(wid * ROWS_PER_W, ROWS_PER_W)])


_bias_sc = functools.partial(
    pl.kernel,
    mesh=plsc.VectorSubcoreMesh(core_axis_name="c", subcore_axis_name="s",
                                num_cores=1),
    out_type=jax.ShapeDtypeStruct((SC_ROWS, D), jnp.float32),
    scratch_types=[
        pltpu.VMEM((ROWS_PER_W, D), jnp.float32),
        pltpu.VMEM((6 * D,), jnp.float32),
        pltpu.SemaphoreType.DMA,
    ],
    compiler_params=pltpu.CompilerParams(skip_device_barrier=True),
)(_bias_body)


def _add_lo_body(x_ref, cht_ref, rgt_ref, bc_ref, wm_ref, bm_ref, wcnt_ref,
                 bcnt_ref, o_ref):
    const = (rgt_ref[0, :] + bc_ref[:] + wm_ref[0, :] + bm_ref[:]
             + wcnt_ref[0, :] + bcnt_ref[:])
    bias = cht_ref[...] + const[None, :]
    o_ref[...] = x_ref[...] + bias[None, :, None, :]


def _add_hi_body(y_ref, x_ref, bias_ref, o_ref):
    del y_ref
    o_ref[...] = x_ref[...] + bias_ref[...][None, :, None, :]


def kernel(x, channel_table, region_table, Wc, bc, Wm, bm, Wcnt, bcnt):
    B, Cx, T, Dx = x.shape
    del Wc

    bias_hi = _bias_sc(channel_table, region_table, bc, Wm, bm, Wcnt, bcnt)

    small = lambda r, c: pl.BlockSpec((r, c), lambda b, cb: (0, 0))
    y = pl.pallas_call(
        _add_lo_body,
        grid=(B, C_SPLIT // CB),
        in_specs=[
            pl.BlockSpec((1, CB, T, Dx), lambda b, cb: (b, cb, 0, 0)),
            pl.BlockSpec((CB, Dx), lambda b, cb: (cb, 0)),
            small(8, Dx),
            pl.BlockSpec((Dx,), lambda b, cb: (0,)),
            small(1, Dx),
            pl.BlockSpec((Dx,), lambda b, cb: (0,)),
            small(1, Dx),
            pl.BlockSpec((Dx,), lambda b, cb: (0,)),
        ],
        out_specs=pl.BlockSpec((1, CB, T, Dx), lambda b, cb: (b, cb, 0, 0)),
        out_shape=jax.ShapeDtypeStruct((B, Cx, T, Dx), x.dtype),
    )(x, channel_table, region_table, bc, Wm, bm, Wcnt, bcnt)

    off = C_SPLIT // CB
    out = pl.pallas_call(
        _add_hi_body,
        grid=(B, (Cx - C_SPLIT) // CB),
        in_specs=[
            pl.BlockSpec((1, 1, 8, 128), lambda b, cb: (0, 0, 0, 0)),
            pl.BlockSpec((1, CB, T, Dx), lambda b, cb: (b, cb + off, 0, 0)),
            pl.BlockSpec((CB, Dx), lambda b, cb: (cb, 0)),
        ],
        out_specs=pl.BlockSpec((1, CB, T, Dx),
                               lambda b, cb: (b, cb + off, 0, 0)),
        out_shape=jax.ShapeDtypeStruct((B, Cx, T, Dx), x.dtype),
        input_output_aliases={0: 0},
    )(y, x, bias_hi)
    return out


# speedup vs baseline: 1.0048x; 1.0048x over previous
import functools

import jax
import jax.numpy as jnp
from jax import lax
from jax.experimental import pallas as pl
from jax.experimental.pallas import tpu as pltpu
from jax.experimental.pallas import tpu_sc as plsc

C, D = 64, 768
CB = 8
C_SPLIT = 16
LANES = 16
NW = 16
SC_ROWS = C - C_SPLIT
ROWS_PER_W = 4
ACTIVE_W = SC_ROWS // ROWS_PER_W


def _bias_body(cht, rgt, bc, wm, bm, wcnt, bcnt, out, rows_v, small_v, sem):
    wid = lax.axis_index("s")

    @pl.when(wid < ACTIVE_W)
    def _():
        _bias_worker(cht, rgt, bc, wm, bm, wcnt, bcnt, out, rows_v, small_v,
                     sem, wid)


def _bias_worker(cht, rgt, bc, wm, bm, wcnt, bcnt, out, rows_v, small_v, sem,
                 wid):
    base = C_SPLIT + wid * ROWS_PER_W
    copies = [
        pltpu.async_copy(cht.at[pl.ds(base, ROWS_PER_W)], rows_v, sem),
        pltpu.async_copy(rgt.at[0], small_v.at[pl.ds(0 * D, D)], sem),
        pltpu.async_copy(bc, small_v.at[pl.ds(1 * D, D)], sem),
        pltpu.async_copy(wm.at[0], small_v.at[pl.ds(2 * D, D)], sem),
        pltpu.async_copy(bm, small_v.at[pl.ds(3 * D, D)], sem),
        pltpu.async_copy(wcnt.at[0], small_v.at[pl.ds(4 * D, D)], sem),
        pltpu.async_copy(bcnt, small_v.at[pl.ds(5 * D, D)], sem),
    ]
    for cp in copies:
        cp.wait()
    for j in range(D // LANES):
        o = j * LANES
        const = (small_v[pl.ds(0 * D + o, LANES)]
                 + small_v[pl.ds(1 * D + o, LANES)]
                 + small_v[pl.ds(2 * D + o, LANES)]
                 + small_v[pl.ds(3 * D + o, LANES)]
                 + small_v[pl.ds(4 * D + o, LANES)]
                 + small_v[pl.ds(5 * D + o, LANES)])
        for r in range(ROWS_PER_W):
            rows_v[r, pl.ds(o, LANES)] = rows_v[r, pl.ds(o, LANES)] + const
    pltpu.sync_copy(rows_v, out.at[pl.ds(wid * ROWS_PER_W, ROWS_PER_W)])


_bias_sc = functools.partial(
    pl.kernel,
    mesh=plsc.VectorSubcoreMesh(core_axis_name="c", subcore_axis_name="s",
                                num_cores=1),
    out_type=jax.ShapeDtypeStruct((SC_ROWS, D), jnp.float32),
    scratch_types=[
        pltpu.VMEM((ROWS_PER_W, D), jnp.float32),
        pltpu.VMEM((6 * D,), jnp.float32),
        pltpu.SemaphoreType.DMA,
    ],
)(_bias_body)


def _add_lo_body(x_ref, cht_ref, rgt_ref, bc_ref, wm_ref, bm_ref, wcnt_ref,
                 bcnt_ref, o_ref):
    const = (rgt_ref[0, :] + bc_ref[:] + wm_ref[0, :] + bm_ref[:]
             + wcnt_ref[0, :] + bcnt_ref[:])
    bias = cht_ref[...] + const[None, :]
    o_ref[...] = x_ref[...] + bias[None, :, None, :]


def _add_hi_body(y_ref, x_ref, bias_ref, o_ref):
    del y_ref
    o_ref[...] = x_ref[...] + bias_ref[...][None, :, None, :]


def kernel(x, channel_table, region_table, Wc, bc, Wm, bm, Wcnt, bcnt):
    B, Cx, T, Dx = x.shape
    del Wc

    bias_hi = _bias_sc(channel_table, region_table, bc, Wm, bm, Wcnt, bcnt)

    small = lambda r, c: pl.BlockSpec((r, c), lambda b, cb: (0, 0))
    y = pl.pallas_call(
        _add_lo_body,
        grid=(B, C_SPLIT // CB),
        in_specs=[
            pl.BlockSpec((1, CB, T, Dx), lambda b, cb: (b, cb, 0, 0)),
            pl.BlockSpec((CB, Dx), lambda b, cb: (cb, 0)),
            small(8, Dx),
            pl.BlockSpec((Dx,), lambda b, cb: (0,)),
            small(1, Dx),
            pl.BlockSpec((Dx,), lambda b, cb: (0,)),
            small(1, Dx),
            pl.BlockSpec((Dx,), lambda b, cb: (0,)),
        ],
        out_specs=pl.BlockSpec((1, CB, T, Dx), lambda b, cb: (b, cb, 0, 0)),
        out_shape=jax.ShapeDtypeStruct((B, Cx, T, Dx), x.dtype),
    )(x, channel_table, region_table, bc, Wm, bm, Wcnt, bcnt)

    off = C_SPLIT // CB
    out = pl.pallas_call(
        _add_hi_body,
        grid=(B, (Cx - C_SPLIT) // CB),
        in_specs=[
            pl.BlockSpec((1, 1, 8, 128), lambda b, cb: (0, 0, 0, 0)),
            pl.BlockSpec((1, CB, T, Dx), lambda b, cb: (b, cb + off, 0, 0)),
            pl.BlockSpec((CB, Dx), lambda b, cb: (cb, 0)),
        ],
        out_specs=pl.BlockSpec((1, CB, T, Dx),
                               lambda b, cb: (b, cb + off, 0, 0)),
        out_shape=jax.ShapeDtypeStruct((B, Cx, T, Dx), x.dtype),
        input_output_aliases={0: 0},
    )(y, x, bias_hi)
    return out
